# baseline (device time: 36839 ns/iter reference)
import jax
import jax.numpy as jnp
from jax import lax
from jax.experimental import pallas as pl
from jax.experimental.pallas import tpu as pltpu

K = 16
KEY_MIN = jnp.iinfo(jnp.int32).min


def _encode(v, iota, n, idx_bits):
    i = lax.bitcast_convert_type(v, jnp.int32)
    key = jnp.where(i < 0, i ^ jnp.int32(0x7FFFFFFF), i)
    return (key & jnp.int32(~((1 << idx_bits) - 1))) | ((n - 1) - iota)


def _decode(key, idx_bits):
    k0 = key & jnp.int32(~((1 << idx_bits) - 1))
    i = jnp.where(k0 < 0, k0 ^ jnp.int32(0x7FFFFFFF), k0)
    return lax.bitcast_convert_type(i, jnp.float32)


def _extract_topk(keys, idx_bits, emit):
    for i in range(K):
        mk = jnp.max(keys, axis=1, keepdims=True)
        emit(i, _decode(mk, idx_bits))
        keys = jnp.where(keys == mk, KEY_MIN, keys)


def kernel(x):
    m, n = x.shape
    nq = n // 4
    qbits = (nq - 1).bit_length()
    mbits = (2 * K - 1).bit_length()

    def body(x_ref, out_ref, xq_ref, send_ref, recv_ref, copy_sem, send_sems, recv_sems):
        my_x = lax.axis_index("x")
        my_y = lax.axis_index("y")
        my_z = lax.axis_index("z")
        partners = [
            (1 - my_x, my_y, my_z),
            (my_x, my_y, 1 - my_z),
            (my_x, 1 - my_y, my_z),
        ]

        q = 2 * my_x + my_z
        copy = pltpu.make_async_copy(
            x_ref.at[:, pl.ds(q * nq, nq)], xq_ref, copy_sem
        )
        copy.start()

        barrier_sem = pltpu.get_barrier_semaphore()
        for nbr in partners:
            pl.semaphore_signal(
                barrier_sem, inc=1, device_id=nbr, device_id_type=pl.DeviceIdType.MESH
            )
        pl.semaphore_wait(barrier_sem, len(partners))
        copy.wait()

        iota = lax.broadcasted_iota(jnp.int32, (m, nq), 1)
        keys = _encode(xq_ref[:, :], iota, nq, qbits)
        _extract_topk(
            keys, qbits, lambda i, v: send_ref.__setitem__((0, slice(None), slice(i, i + 1)), v)
        )

        iota2 = lax.broadcasted_iota(jnp.int32, (m, 2 * K), 1)
        for s, nbr in enumerate(partners):
            rdma = pltpu.make_async_remote_copy(
                src_ref=send_ref.at[s],
                dst_ref=recv_ref.at[s],
                send_sem=send_sems.at[s],
                recv_sem=recv_sems.at[s],
                device_id=nbr,
                device_id_type=pl.DeviceIdType.MESH,
            )
            rdma.start()
            rdma.wait()

            comb = jnp.concatenate([send_ref[s], recv_ref[s]], axis=1)
            kc = _encode(comb, iota2, 2 * K, mbits)
            if s < 2:
                emit = lambda i, v, _s=s: send_ref.__setitem__(
                    (_s + 1, slice(None), slice(i, i + 1)), v
                )
            else:
                emit = lambda i, v: out_ref.__setitem__(
                    (slice(None), slice(i, i + 1)), v
                )
            _extract_topk(kc, mbits, emit)

    return pl.pallas_call(
        body,
        out_shape=jax.ShapeDtypeStruct((m, K), jnp.float32),
        in_specs=[pl.BlockSpec(memory_space=pl.MemorySpace.ANY)],
        out_specs=pl.BlockSpec(memory_space=pltpu.VMEM),
        scratch_shapes=[
            pltpu.VMEM((m, nq), jnp.float32),
            pltpu.VMEM((3, m, K), jnp.float32),
            pltpu.VMEM((3, m, K), jnp.float32),
            pltpu.SemaphoreType.DMA,
            pltpu.SemaphoreType.DMA((3,)),
            pltpu.SemaphoreType.DMA((3,)),
        ],
        compiler_params=pltpu.CompilerParams(collective_id=0),
    )(x)


# device time: 32824 ns/iter; 1.1223x vs baseline; 1.1223x over previous
import jax
import jax.numpy as jnp
from jax import lax
from jax.experimental import pallas as pl
from jax.experimental.pallas import tpu as pltpu

K = 16
KEY_MIN = jnp.iinfo(jnp.int32).min


def _encode(v, iota, n, idx_bits):
    i = lax.bitcast_convert_type(v, jnp.int32)
    key = jnp.where(i < 0, i ^ jnp.int32(0x7FFFFFFF), i)
    return (key & jnp.int32(~((1 << idx_bits) - 1))) | ((n - 1) - iota)


def _decode(key, idx_bits):
    k0 = key & jnp.int32(~((1 << idx_bits) - 1))
    i = jnp.where(k0 < 0, k0 ^ jnp.int32(0x7FFFFFFF), k0)
    return lax.bitcast_convert_type(i, jnp.float32)


def _bitonic_sort16(s, iota_k, descending):
    for d in (8, 4, 2, 1):
        low = (iota_k & d) == 0
        p = jnp.where(low, pltpu.roll(s, K - d, 1), pltpu.roll(s, d, 1))
        big, small = jnp.maximum(s, p), jnp.minimum(s, p)
        s = jnp.where(low, big, small) if descending else jnp.where(low, small, big)
    return s


def kernel(x):
    m, n = x.shape
    nq = n // 4
    qbits = (nq - 1).bit_length()

    def body(x_ref, out_ref, xq_ref, send_ref, recv_ref, copy_sem, send_sems, recv_sems):
        my_x = lax.axis_index("x")
        my_y = lax.axis_index("y")
        my_z = lax.axis_index("z")
        partners = [
            (1 - my_x, my_y, my_z),
            (my_x, my_y, 1 - my_z),
            (my_x, 1 - my_y, my_z),
        ]

        q = 2 * my_x + my_z
        copy = pltpu.make_async_copy(x_ref.at[:, pl.ds(q * nq, nq)], xq_ref, copy_sem)
        copy.start()

        barrier_sem = pltpu.get_barrier_semaphore()
        for nbr in partners:
            pl.semaphore_signal(
                barrier_sem, inc=1, device_id=nbr, device_id_type=pl.DeviceIdType.MESH
            )
        pl.semaphore_wait(barrier_sem, len(partners))
        copy.wait()

        iota = lax.broadcasted_iota(jnp.int32, (m, nq), 1)
        keys = _encode(xq_ref[:, :], iota, nq, qbits)
        iota_k = lax.broadcasted_iota(jnp.int32, (m, K), 1)
        acc_d = jnp.full((m, K), KEY_MIN, jnp.int32)
        acc_a = jnp.full((m, K), KEY_MIN, jnp.int32)
        for i in range(K):
            mk = jnp.max(keys, axis=1, keepdims=True)
            acc_d = jnp.where(iota_k == i, mk, acc_d)
            acc_a = jnp.where(iota_k == K - 1 - i, mk, acc_a)
            keys = jnp.where(keys == mk, KEY_MIN, keys)
        a_desc = _decode(acc_d, qbits)
        send_ref[0] = _decode(acc_a, qbits)

        for s, nbr in enumerate(partners):
            rdma = pltpu.make_async_remote_copy(
                src_ref=send_ref.at[s],
                dst_ref=recv_ref.at[s],
                send_sem=send_sems.at[s],
                recv_sem=recv_sems.at[s],
                device_id=nbr,
                device_id_type=pl.DeviceIdType.MESH,
            )
            rdma.start()
            rdma.wait()

            top = jnp.maximum(a_desc, recv_ref[s])
            a_desc = _bitonic_sort16(top, iota_k, descending=True)
            if s < 2:
                send_ref[s + 1] = _bitonic_sort16(top, iota_k, descending=False)
        out_ref[:, :] = a_desc

    return pl.pallas_call(
        body,
        out_shape=jax.ShapeDtypeStruct((m, K), jnp.float32),
        in_specs=[pl.BlockSpec(memory_space=pl.MemorySpace.ANY)],
        out_specs=pl.BlockSpec(memory_space=pltpu.VMEM),
        scratch_shapes=[
            pltpu.VMEM((m, nq), jnp.float32),
            pltpu.VMEM((3, m, K), jnp.float32),
            pltpu.VMEM((3, m, K), jnp.float32),
            pltpu.SemaphoreType.DMA,
            pltpu.SemaphoreType.DMA((3,)),
            pltpu.SemaphoreType.DMA((3,)),
        ],
        compiler_params=pltpu.CompilerParams(collective_id=0),
    )(x)


# device time: 30223 ns/iter; 1.2189x vs baseline; 1.0861x over previous
import jax
import jax.numpy as jnp
from jax import lax
from jax.experimental import pallas as pl
from jax.experimental.pallas import tpu as pltpu

K = 16
KEY_MIN = jnp.iinfo(jnp.int32).min


def _encode(v, iota, n, idx_bits):
    i = lax.bitcast_convert_type(v, jnp.int32)
    key = jnp.where(i < 0, i ^ jnp.int32(0x7FFFFFFF), i)
    return (key & jnp.int32(~((1 << idx_bits) - 1))) | ((n - 1) - iota)


def _decode(key, idx_bits):
    k0 = key & jnp.int32(~((1 << idx_bits) - 1))
    i = jnp.where(k0 < 0, k0 ^ jnp.int32(0x7FFFFFFF), k0)
    return lax.bitcast_convert_type(i, jnp.float32)


def _bitonic_sort16(s, iota_k, descending):
    for d in (8, 4, 2, 1):
        low = (iota_k & d) == 0
        p = jnp.where(low, pltpu.roll(s, K - d, 1), pltpu.roll(s, d, 1))
        big, small = jnp.maximum(s, p), jnp.minimum(s, p)
        s = jnp.where(low, big, small) if descending else jnp.where(low, small, big)
    return s


def kernel(x):
    m, n = x.shape
    nq = n // 4
    qbits = (nq - 1).bit_length()

    def body(x_ref, out_ref, xq_ref, sbuf, rbuf1, rbuf2, rbuf3,
             copy_sem, send_sems, recv_sems):
        my_x = lax.axis_index("x")
        my_y = lax.axis_index("y")
        my_z = lax.axis_index("z")
        x_nbr = (1 - my_x, my_y, my_z)
        y_nbr = (my_x, 1 - my_y, my_z)
        z_nbr = (my_x, my_y, 1 - my_z)

        q = 2 * my_x + my_z
        copy = pltpu.make_async_copy(x_ref.at[:, pl.ds(q * nq, nq)], xq_ref, copy_sem)
        copy.start()

        barrier_sem = pltpu.get_barrier_semaphore()
        for nbr in (x_nbr, y_nbr, z_nbr):
            pl.semaphore_signal(
                barrier_sem, inc=1, device_id=nbr, device_id_type=pl.DeviceIdType.MESH
            )
        copy.wait()

        iota = lax.broadcasted_iota(jnp.int32, (m, nq), 1)
        keys = _encode(xq_ref[:, :], iota, nq, qbits)
        iota_k = lax.broadcasted_iota(jnp.int32, (m, K), 1)
        acc_d = jnp.full((m, K), KEY_MIN, jnp.int32)
        acc_a = jnp.full((m, K), KEY_MIN, jnp.int32)
        for i in range(K):
            mk = jnp.max(keys, axis=1, keepdims=True)
            acc_d = jnp.where(iota_k == i, mk, acc_d)
            acc_a = jnp.where(iota_k == K - 1 - i, mk, acc_a)
            keys = jnp.where(keys == mk, KEY_MIN, keys)
        a_desc = _decode(acc_d, qbits)
        sbuf[:, :] = _decode(acc_a, qbits)

        pl.semaphore_wait(barrier_sem, 3)

        def exchange(idx, nbr, src, dst):
            r = pltpu.make_async_remote_copy(
                src_ref=src, dst_ref=dst,
                send_sem=send_sems.at[idx], recv_sem=recv_sems.at[idx],
                device_id=nbr, device_id_type=pl.DeviceIdType.MESH,
            )
            r.start()
            return r

        r1x = exchange(0, x_nbr, sbuf, rbuf1.at[0])
        r1y = exchange(1, y_nbr, sbuf, rbuf1.at[1])
        r1z = exchange(2, z_nbr, sbuf, rbuf1.at[2])

        r1x.wait_recv()
        r2y = exchange(4, y_nbr, rbuf1.at[0], rbuf2.at[1])
        r1y.wait_recv()
        r2z = exchange(5, z_nbr, rbuf1.at[1], rbuf2.at[2])
        r1z.wait_recv()
        r2x = exchange(3, x_nbr, rbuf1.at[2], rbuf2.at[0])

        r2z.wait_recv()
        r3 = exchange(6, x_nbr, rbuf2.at[2], rbuf3)

        def merge(acc, b_asc):
            return _bitonic_sort16(jnp.maximum(acc, b_asc), iota_k, descending=True)

        for i in range(3):
            a_desc = merge(a_desc, rbuf1[i])
        r2x.wait_recv()
        a_desc = merge(a_desc, rbuf2[0])
        r2y.wait_recv()
        a_desc = merge(a_desc, rbuf2[1])
        a_desc = merge(a_desc, rbuf2[2])
        r3.wait_recv()
        out_ref[:, :] = merge(a_desc, rbuf3[:, :])

        for r in (r1x, r1y, r1z, r2x, r2y, r2z, r3):
            r.wait_send()

    return pl.pallas_call(
        body,
        out_shape=jax.ShapeDtypeStruct((m, K), jnp.float32),
        in_specs=[pl.BlockSpec(memory_space=pl.MemorySpace.ANY)],
        out_specs=pl.BlockSpec(memory_space=pltpu.VMEM),
        scratch_shapes=[
            pltpu.VMEM((m, nq), jnp.float32),
            pltpu.VMEM((m, K), jnp.float32),
            pltpu.VMEM((3, m, K), jnp.float32),
            pltpu.VMEM((3, m, K), jnp.float32),
            pltpu.VMEM((m, K), jnp.float32),
            pltpu.SemaphoreType.DMA,
            pltpu.SemaphoreType.DMA((7,)),
            pltpu.SemaphoreType.DMA((7,)),
        ],
        compiler_params=pltpu.CompilerParams(collective_id=0),
    )(x)
